# baseline (device time: 23403 ns/iter reference)
import jax
import jax.numpy as jnp
from jax import lax
from jax.experimental import pallas as pl
from jax.experimental.pallas import tpu as pltpu

N_DEV = 16
B = 2
S = 128
HQ = 4
DH = 64
NEG = -1e9


def kernel(x, Wq, K_ext, V_ext, Wo):
    d_model = x.shape[-1]

    def body(x_ref, wq_ref, k_ref, v_ref, wo_ref, out_ref,
             kbuf, vbuf, ctx_ref, send_sems, recv_sems):
        my = lax.axis_index("i")
        left = my - 1
        right = my + 1
        has_left = my > 0
        has_right = my < N_DEV - 1

        @pl.when(jnp.logical_not(has_left))
        def _():
            kbuf[0] = jnp.zeros((B, S, HQ, DH), jnp.float32)
            vbuf[0] = jnp.zeros((B, S, HQ, DH), jnp.float32)

        @pl.when(jnp.logical_not(has_right))
        def _():
            kbuf[1] = jnp.zeros((B, S, HQ, DH), jnp.float32)
            vbuf[1] = jnp.zeros((B, S, HQ, DH), jnp.float32)

        barrier_sem = pltpu.get_barrier_semaphore()

        @pl.when(has_left)
        def _():
            pl.semaphore_signal(barrier_sem, inc=1, device_id=(left,),
                                device_id_type=pl.DeviceIdType.MESH)

        @pl.when(has_right)
        def _():
            pl.semaphore_signal(barrier_sem, inc=1, device_id=(right,),
                                device_id_type=pl.DeviceIdType.MESH)

        n_nbr = has_left.astype(jnp.int32) + has_right.astype(jnp.int32)
        pl.semaphore_wait(barrier_sem, n_nbr)

        def halo_rdma(src, buf, slot, sem, nbr):
            return pltpu.make_async_remote_copy(
                src_ref=src,
                dst_ref=buf.at[slot],
                send_sem=send_sems.at[sem],
                recv_sem=recv_sems.at[sem],
                device_id=(nbr,),
                device_id_type=pl.DeviceIdType.MESH,
            )

        @pl.when(has_right)
        def _():
            halo_rdma(k_ref, kbuf, 0, 0, right).start()
            halo_rdma(v_ref, vbuf, 0, 1, right).start()

        @pl.when(has_left)
        def _():
            halo_rdma(k_ref, kbuf, 1, 2, left).start()
            halo_rdma(v_ref, vbuf, 1, 3, left).start()

        q_all = [
            jnp.dot(x_ref[b], wq_ref[...], preferred_element_type=jnp.float32)
            for b in range(B)
        ]

        @pl.when(has_left)
        def _():
            halo_rdma(k_ref, kbuf, 0, 0, left).wait_recv()
            halo_rdma(v_ref, vbuf, 0, 1, left).wait_recv()

        @pl.when(has_right)
        def _():
            halo_rdma(k_ref, kbuf, 1, 2, right).wait_recv()
            halo_rdma(v_ref, vbuf, 1, 3, right).wait_recv()

        qi = lax.broadcasted_iota(jnp.int32, (S, S), 0)
        kj = lax.broadcasted_iota(jnp.int32, (S, S), 1)
        mask_l = jnp.logical_and(qi <= kj, has_left)
        mask_r = jnp.logical_and(qi >= kj, has_right)

        dn = (((1,), (1,)), ((), ()))
        for b in range(B):
            for h in range(HQ):
                q = q_all[b][:, h * DH:(h + 1) * DH] * 0.125
                s_l = lax.dot_general(q, kbuf[0, b, :, h, :], dn,
                                      preferred_element_type=jnp.float32)
                s_c = lax.dot_general(q, k_ref[b, :, h, :], dn,
                                      preferred_element_type=jnp.float32)
                s_r = lax.dot_general(q, kbuf[1, b, :, h, :], dn,
                                      preferred_element_type=jnp.float32)
                s_l = jnp.where(mask_l, s_l, NEG)
                s_r = jnp.where(mask_r, s_r, NEG)
                s = jnp.concatenate([s_l, s_c, s_r], axis=1)
                m = jnp.max(s, axis=1, keepdims=True)
                w = jnp.exp(s - m)
                w = w / jnp.sum(w, axis=1, keepdims=True)
                v_all = jnp.concatenate(
                    [vbuf[0, b, :, h, :], v_ref[b, :, h, :],
                     vbuf[1, b, :, h, :]], axis=0)
                ctx_ref[b, h] = jnp.dot(w, v_all,
                                        preferred_element_type=jnp.float32)

        for b in range(B):
            ctx_b = jnp.concatenate([ctx_ref[b, h] for h in range(HQ)],
                                    axis=1)
            out_ref[b] = jnp.dot(ctx_b, wo_ref[...],
                                 preferred_element_type=jnp.float32)

        @pl.when(has_right)
        def _():
            halo_rdma(k_ref, kbuf, 0, 0, right).wait_send()
            halo_rdma(v_ref, vbuf, 0, 1, right).wait_send()

        @pl.when(has_left)
        def _():
            halo_rdma(k_ref, kbuf, 1, 2, left).wait_send()
            halo_rdma(v_ref, vbuf, 1, 3, left).wait_send()

    return pl.pallas_call(
        body,
        out_shape=jax.ShapeDtypeStruct((B, S, d_model), jnp.float32),
        in_specs=[pl.BlockSpec(memory_space=pltpu.VMEM)] * 5,
        out_specs=pl.BlockSpec(memory_space=pltpu.VMEM),
        scratch_shapes=[
            pltpu.VMEM((2, B, S, HQ, DH), jnp.float32),
            pltpu.VMEM((2, B, S, HQ, DH), jnp.float32),
            pltpu.VMEM((B, HQ, S, DH), jnp.float32),
            pltpu.SemaphoreType.DMA((4,)),
            pltpu.SemaphoreType.DMA((4,)),
        ],
        compiler_params=pltpu.CompilerParams(collective_id=0),
    )(x, Wq, K_ext, V_ext, Wo)


# device time: 8077 ns/iter; 2.8975x vs baseline; 2.8975x over previous
import jax
import jax.numpy as jnp
from jax import lax
from jax.experimental import pallas as pl
from jax.experimental.pallas import tpu as pltpu

N_DEV = 16
B = 2
S = 128
HQ = 4
DH = 64
NEG = -1e9


def kernel(x, Wq, K_ext, V_ext, Wo):
    d_model = x.shape[-1]

    def body(x_ref, wq_ref, k_ref, v_ref, wo_ref, out_ref,
             kbuf, vbuf, send_sems, recv_sems):
        my = lax.axis_index("i")
        left = my - 1
        right = my + 1
        has_left = my > 0
        has_right = my < N_DEV - 1

        @pl.when(jnp.logical_not(has_left))
        def _():
            kbuf[0] = jnp.zeros((B, S, HQ, DH), jnp.float32)
            vbuf[0] = jnp.zeros((B, S, HQ, DH), jnp.float32)

        @pl.when(jnp.logical_not(has_right))
        def _():
            kbuf[1] = jnp.zeros((B, S, HQ, DH), jnp.float32)
            vbuf[1] = jnp.zeros((B, S, HQ, DH), jnp.float32)

        barrier_sem = pltpu.get_barrier_semaphore()

        @pl.when(has_left)
        def _():
            pl.semaphore_signal(barrier_sem, inc=1, device_id=(left,),
                                device_id_type=pl.DeviceIdType.MESH)

        @pl.when(has_right)
        def _():
            pl.semaphore_signal(barrier_sem, inc=1, device_id=(right,),
                                device_id_type=pl.DeviceIdType.MESH)

        n_nbr = has_left.astype(jnp.int32) + has_right.astype(jnp.int32)
        pl.semaphore_wait(barrier_sem, n_nbr)

        def halo_rdma(src, buf, slot, sem, nbr):
            return pltpu.make_async_remote_copy(
                src_ref=src,
                dst_ref=buf.at[slot],
                send_sem=send_sems.at[sem],
                recv_sem=recv_sems.at[sem],
                device_id=(nbr,),
                device_id_type=pl.DeviceIdType.MESH,
            )

        @pl.when(has_right)
        def _():
            halo_rdma(k_ref, kbuf, 0, 0, right).start()
            halo_rdma(v_ref, vbuf, 0, 1, right).start()

        @pl.when(has_left)
        def _():
            halo_rdma(k_ref, kbuf, 1, 2, left).start()
            halo_rdma(v_ref, vbuf, 1, 3, left).start()

        x2 = x_ref[...].reshape(B * S, x_ref.shape[-1])
        q2 = jnp.dot(x2, wq_ref[...],
                     preferred_element_type=jnp.float32) * 0.125

        qi = lax.broadcasted_iota(jnp.int32, (S, S), 0)
        kj = lax.broadcasted_iota(jnp.int32, (S, S), 1)
        full_mask = jnp.concatenate(
            [jnp.logical_and(qi <= kj, has_left),
             jnp.ones((S, S), jnp.bool_),
             jnp.logical_and(qi >= kj, has_right)], axis=1)

        @pl.when(has_left)
        def _():
            halo_rdma(k_ref, kbuf, 0, 0, left).wait_recv()
            halo_rdma(v_ref, vbuf, 0, 1, left).wait_recv()

        @pl.when(has_right)
        def _():
            halo_rdma(k_ref, kbuf, 1, 2, right).wait_recv()
            halo_rdma(v_ref, vbuf, 1, 3, right).wait_recv()

        dn = (((1,), (1,)), ((), ()))
        ctx_rows = []
        for b in range(B):
            heads = []
            for h in range(HQ):
                q = q2[b * S:(b + 1) * S, h * DH:(h + 1) * DH]
                k_all = jnp.concatenate(
                    [kbuf[0, b, :, h, :], k_ref[b, :, h, :],
                     kbuf[1, b, :, h, :]], axis=0)
                s = lax.dot_general(q, k_all, dn,
                                    preferred_element_type=jnp.float32)
                s = jnp.where(full_mask, s, NEG)
                m = jnp.max(s, axis=1, keepdims=True)
                w = jnp.exp(s - m)
                w = w / jnp.sum(w, axis=1, keepdims=True)
                v_all = jnp.concatenate(
                    [vbuf[0, b, :, h, :], v_ref[b, :, h, :],
                     vbuf[1, b, :, h, :]], axis=0)
                heads.append(jnp.dot(w, v_all,
                                     preferred_element_type=jnp.float32))
            ctx_rows.append(jnp.concatenate(heads, axis=1))
        ctx = jnp.concatenate(ctx_rows, axis=0)
        out = jnp.dot(ctx, wo_ref[...], preferred_element_type=jnp.float32)
        for b in range(B):
            out_ref[b] = out[b * S:(b + 1) * S, :]

        @pl.when(has_right)
        def _():
            halo_rdma(k_ref, kbuf, 0, 0, right).wait_send()
            halo_rdma(v_ref, vbuf, 0, 1, right).wait_send()

        @pl.when(has_left)
        def _():
            halo_rdma(k_ref, kbuf, 1, 2, left).wait_send()
            halo_rdma(v_ref, vbuf, 1, 3, left).wait_send()

    return pl.pallas_call(
        body,
        out_shape=jax.ShapeDtypeStruct((B, S, d_model), jnp.float32),
        in_specs=[pl.BlockSpec(memory_space=pltpu.VMEM)] * 5,
        out_specs=pl.BlockSpec(memory_space=pltpu.VMEM),
        scratch_shapes=[
            pltpu.VMEM((2, B, S, HQ, DH), jnp.float32),
            pltpu.VMEM((2, B, S, HQ, DH), jnp.float32),
            pltpu.SemaphoreType.DMA((4,)),
            pltpu.SemaphoreType.DMA((4,)),
        ],
        compiler_params=pltpu.CompilerParams(collective_id=0),
    )(x, Wq, K_ext, V_ext, Wo)
